# baseline (speedup 1.0000x reference)
import functools

import jax
import jax.numpy as jnp
from jax import lax
from jax.experimental import pallas as pl
from jax.experimental.pallas import tpu as pltpu
from jax.experimental.pallas import tpu_sc as plsc

VOCAB = 100000
EMBED_DIM = 64
BATCH = 1024
SEQ = 200

_NC = 2
_NS = 16
_NW = _NC * _NS
_ROWS_PER_W = BATCH // _NW
_C0 = 128
_C1 = SEQ - _C0


def _sc_pool_body(x_hbm, table_hbm, out_hbm, idx_v, rows_a, rows_b,
                  pooled_v, sem_a, sem_b):
    wid = lax.axis_index("s") * _NC + lax.axis_index("c")
    base = wid * _ROWS_PER_W

    pltpu.sync_copy(x_hbm.at[pl.ds(base, _ROWS_PER_W)], idx_v)

    inv = jnp.float32(1.0 / SEQ)

    def fire(i, rows, sem):
        pltpu.async_copy(
            table_hbm.at[idx_v.at[i, pl.ds(0, _C0)]],
            rows.at[pl.ds(0, _C0)], sem)
        pltpu.async_copy(
            table_hbm.at[idx_v.at[i, pl.ds(_C0, _C1)]],
            rows.at[pl.ds(_C0, _C1)], sem)

    def drain(rows, sem):
        pltpu.make_async_copy(
            table_hbm.at[pl.ds(0, _C0)], rows.at[pl.ds(0, _C0)], sem).wait()
        pltpu.make_async_copy(
            table_hbm.at[pl.ds(0, _C1)], rows.at[pl.ds(_C0, _C1)], sem).wait()

    def reduce_row(i, rows):
        def acc_body(j, accs):
            a0, a1, a2, a3 = accs
            return (a0 + rows[j, pl.ds(0, 16)],
                    a1 + rows[j, pl.ds(16, 16)],
                    a2 + rows[j, pl.ds(32, 16)],
                    a3 + rows[j, pl.ds(48, 16)])

        z = jnp.zeros((16,), jnp.float32)
        a0, a1, a2, a3 = lax.fori_loop(0, SEQ, acc_body, (z, z, z, z))
        pooled_v[i, pl.ds(0, 16)] = a0 * inv
        pooled_v[i, pl.ds(16, 16)] = a1 * inv
        pooled_v[i, pl.ds(32, 16)] = a2 * inv
        pooled_v[i, pl.ds(48, 16)] = a3 * inv

    fire(jnp.int32(0), rows_a, sem_a)

    def pair_body(p, carry):
        del carry
        ia = jnp.int32(2) * p
        fire(ia + 1, rows_b, sem_b)
        drain(rows_a, sem_a)
        reduce_row(ia, rows_a)

        @pl.when(ia + 2 < _ROWS_PER_W)
        def _():
            fire(ia + 2, rows_a, sem_a)

        drain(rows_b, sem_b)
        reduce_row(ia + 1, rows_b)
        return 0

    lax.fori_loop(jnp.int32(0), jnp.int32(_ROWS_PER_W // 2), pair_body, 0)

    pltpu.sync_copy(pooled_v, out_hbm.at[pl.ds(base, _ROWS_PER_W)])


@functools.cache
def _build_sc_pool():
    return pl.kernel(
        _sc_pool_body,
        mesh=plsc.VectorSubcoreMesh(
            core_axis_name="c", subcore_axis_name="s",
            num_cores=_NC, num_subcores=_NS),
        out_type=jax.ShapeDtypeStruct((BATCH, EMBED_DIM), jnp.float32),
        scratch_types=[
            pltpu.VMEM((_ROWS_PER_W, SEQ), jnp.int32),
            pltpu.VMEM((SEQ, EMBED_DIM), jnp.float32),
            pltpu.VMEM((SEQ, EMBED_DIM), jnp.float32),
            pltpu.VMEM((_ROWS_PER_W, EMBED_DIM), jnp.float32),
            pltpu.SemaphoreType.DMA,
            pltpu.SemaphoreType.DMA,
        ],
        compiler_params=pltpu.CompilerParams(use_tc_tiling_on_sc=False),
    )


_N_BLK = 6144


def _i32(v):
    return jnp.asarray(v, jnp.int32)


def _mm_body(p_ref, w_ref, b_ref, o_ref):
    acc = lax.dot_general(
        p_ref[...], w_ref[...],
        (((1,), (1,)), ((), ())),
        preferred_element_type=jnp.float32)
    o_ref[...] = acc + b_ref[...]


def _projection(pooled, W, b2d):
    grid = (pl.cdiv(VOCAB, _N_BLK),)
    return pl.pallas_call(
        _mm_body,
        grid=grid,
        in_specs=[
            pl.BlockSpec((BATCH, EMBED_DIM), lambda j: (_i32(0), _i32(0))),
            pl.BlockSpec((_N_BLK, EMBED_DIM), lambda j: (_i32(j), _i32(0))),
            pl.BlockSpec((1, _N_BLK), lambda j: (_i32(0), _i32(j))),
        ],
        out_specs=pl.BlockSpec((BATCH, _N_BLK), lambda j: (_i32(0), _i32(j))),
        out_shape=jax.ShapeDtypeStruct((BATCH, VOCAB), jnp.float32),
    )(pooled, W, b2d)


def kernel(x, table, W, b):
    x32 = x.astype(jnp.int32)
    pooled = _build_sc_pool()(x32, table)
    return _projection(pooled, W, b.reshape(1, VOCAB))
